# baseline (speedup 1.0000x reference)
import functools

import jax
import jax.numpy as jnp
from jax import lax
from jax.experimental import pallas as pl
from jax.experimental.pallas import tpu as pltpu
from jax.experimental.pallas import tpu_sc as plsc

_N_TOKENS = 8192
_N_CODES = 8192
_D = 64
_BETA = 0.25

_T = 256
_GRID = _N_TOKENS // _T


def _tc_body(z_ref, e_ref, idx_ref, loss_ref):
    z = z_ref[...]
    e = e_ref[...]
    znorm = jnp.sum(z * z, axis=1, keepdims=True)
    enorm = jnp.sum(e * e, axis=1)
    mm = lax.dot_general(z, e, (((1,), (1,)), ((), ())),
                         preferred_element_type=jnp.float32)
    d = (znorm + enorm[None, :]) - 2.0 * mm
    dmin = jnp.min(d, axis=1)
    iota = lax.broadcasted_iota(jnp.int32, d.shape, 1)
    masked = jnp.where(d == dmin[:, None], iota, _N_CODES)
    idx = jnp.min(masked, axis=1)
    idx_ref[0, 0, :] = idx

    @pl.when(pl.program_id(0) == 0)
    def _init():
        loss_ref[0, 0] = 0.0

    loss_ref[0, 0] += jnp.sum(dmin)

    @pl.when(pl.program_id(0) == _GRID - 1)
    def _finish():
        s = loss_ref[0, 0] / jnp.float32(_N_TOKENS * _D)
        loss_ref[0, 0] = _BETA * s + s


def _tc_argmin(z_flat, embedding):
    return pl.pallas_call(
        _tc_body,
        grid=(_GRID,),
        in_specs=[
            pl.BlockSpec((_T, _D), lambda i: (i, 0)),
            pl.BlockSpec((_N_CODES, _D), lambda i: (0, 0)),
        ],
        out_specs=[
            pl.BlockSpec((1, 1, _T), lambda i: (i, 0, 0)),
            pl.BlockSpec((1, 1), lambda i: (0, 0),
                         memory_space=pltpu.MemorySpace.SMEM),
        ],
        out_shape=[
            jax.ShapeDtypeStruct((_GRID, 1, _T), jnp.int32),
            jax.ShapeDtypeStruct((1, 1), jnp.float32),
        ],
    )(z_flat, embedding)


@functools.lru_cache(maxsize=None)
def _sc_gather_st():
    info = plsc.get_sparse_core_info()
    nw = info.num_cores * info.num_subcores
    b_per_w = _N_TOKENS // nw

    def _sc_body(e_hbm, idx_hbm, zp_hbm, out_hbm, idx_v, rows_v, zp_v, sem):
        wid = lax.axis_index("s") * info.num_cores + lax.axis_index("c")
        base = wid * b_per_w
        pltpu.sync_copy(idx_hbm.at[pl.ds(base, b_per_w)], idx_v)
        pltpu.async_copy(e_hbm.at[idx_v], rows_v, sem).wait()
        pltpu.sync_copy(zp_hbm.at[pl.ds(base, b_per_w)], zp_v)

        def _row(r, _):
            for k in range(_D // 16):
                sl = pl.ds(k * 16, 16)
                zq = rows_v[r, sl]
                zp = zp_v[r, sl]
                rows_v[r, sl] = zp + (zq - zp)
            return 0

        lax.fori_loop(0, b_per_w, _row, 0)
        pltpu.sync_copy(rows_v, out_hbm.at[pl.ds(base, b_per_w)])

    return pl.kernel(
        _sc_body,
        out_type=jax.ShapeDtypeStruct((_N_TOKENS, _D), jnp.float32),
        mesh=plsc.VectorSubcoreMesh(core_axis_name="c", subcore_axis_name="s"),
        scratch_types=[
            pltpu.VMEM((b_per_w,), jnp.int32),
            pltpu.VMEM((b_per_w, _D), jnp.float32),
            pltpu.VMEM((b_per_w, _D), jnp.float32),
            pltpu.SemaphoreType.DMA,
        ],
        compiler_params=pltpu.CompilerParams(use_tc_tiling_on_sc=False),
    )


def kernel(z, embedding):
    zp = jnp.transpose(z, (0, 2, 3, 1))
    z_flat = zp.reshape(_N_TOKENS, _D)
    idx3, loss = _tc_argmin(z_flat, embedding)
    indices = idx3.reshape(_N_TOKENS)
    zq_flat = _sc_gather_st()(embedding, indices, z_flat)
    z_q_out = jnp.transpose(zq_flat.reshape(zp.shape), (0, 3, 1, 2))
    return z_q_out, loss[0, 0], indices
